# baseline (speedup 1.0000x reference)
import functools

import jax
import jax.numpy as jnp
from jax import lax
from jax.experimental import pallas as pl
from jax.experimental.pallas import tpu as pltpu
from jax.experimental.pallas import tpu_sc as plsc

D = 32
SEQ = 200
NBUF = 8


def _make_gather(batch: int):
  info = plsc.get_sparse_core_info()
  nw = info.num_cores * info.num_subcores
  b_per_w = batch // nw
  nchunk = 2 * b_per_w
  ngroup = nchunk // NBUF
  assert b_per_w * nw == batch and ngroup * NBUF == nchunk
  assert ngroup % 2 == 0 and NBUF % 2 == 0

  mesh = plsc.VectorSubcoreMesh(core_axis_name="c", subcore_axis_name="s")

  @functools.partial(
      pl.kernel,
      out_type=jax.ShapeDtypeStruct((batch, SEQ, D), jnp.float32),
      mesh=mesh,
      compiler_params=pltpu.CompilerParams(use_tc_tiling_on_sc=False),
      scratch_types=[
          pltpu.VMEM((b_per_w, SEQ), jnp.int32),
          pltpu.VMEM((2 * NBUF, 128, D), jnp.float32),
          pltpu.SemaphoreType.DMA,
          pltpu.SemaphoreType.DMA,
          pltpu.SemaphoreType.DMA,
          pltpu.SemaphoreType.DMA,
      ],
  )
  def gather_kernel(table_hbm, idx_hbm, out_hbm, idx_v, rows_v,
                    gsem0, gsem1, wsem0, wsem1):
    wid = lax.axis_index("s") * info.num_cores + lax.axis_index("c")
    b_base = wid * b_per_w
    gsem = (gsem0, gsem1)
    wsem = (wsem0, wsem1)

    pltpu.sync_copy(idx_hbm.at[wid], idx_v)

    def chunk_refs(half, g, b):
      j = g * (NBUF // 2) + b // 2
      buf = half * NBUF + b
      if b % 2 == 0:
        idx_sl = idx_v.at[j, pl.ds(0, 128)]
        row_sl = rows_v.at[buf]
        out_sl = out_hbm.at[b_base + j, pl.ds(0, 128)]
      else:
        idx_sl = idx_v.at[j, pl.ds(128, 72)]
        row_sl = rows_v.at[buf, pl.ds(0, 72)]
        out_sl = out_hbm.at[b_base + j, pl.ds(128, 72)]
      return idx_sl, row_sl, out_sl

    def fire_gathers(half, g):
      for b in range(NBUF):
        idx_sl, row_sl, _ = chunk_refs(half, g, b)
        pltpu.async_copy(table_hbm.at[idx_sl], row_sl, gsem[half])

    def wait_gathers(half, g):
      for b in range(NBUF):
        idx_sl, row_sl, _ = chunk_refs(half, g, b)
        pltpu.make_async_copy(table_hbm.at[idx_sl], row_sl,
                              gsem[half]).wait()

    def fire_writes(half, g):
      for b in range(NBUF):
        _, row_sl, out_sl = chunk_refs(half, g, b)
        pltpu.async_copy(row_sl, out_sl, wsem[half])

    def wait_writes(half, g):
      for b in range(NBUF):
        _, row_sl, out_sl = chunk_refs(half, g, b)
        pltpu.make_async_copy(row_sl, out_sl, wsem[half]).wait()

    fire_gathers(0, 0)

    @pl.loop(0, ngroup, step=2)
    def body(g):
      wait_gathers(0, g)
      fire_writes(0, g)

      @pl.when(g > 0)
      def _():
        wait_writes(1, g - 1)

      fire_gathers(1, g + 1)

      wait_gathers(1, g + 1)
      fire_writes(1, g + 1)

      @pl.when(g + 2 < ngroup)
      def _():
        wait_writes(0, g)
        fire_gathers(0, g + 2)

    wait_writes(0, ngroup - 2)
    wait_writes(1, ngroup - 1)

  return gather_kernel


def kernel(news_tokens, embedding_table):
  batch, seq = news_tokens.shape
  info = plsc.get_sparse_core_info()
  nw = info.num_cores * info.num_subcores
  idx = news_tokens.astype(jnp.int32).reshape(nw, batch // nw, seq)
  return _make_gather(batch)(embedding_table, idx)


# speedup vs baseline: 2.4190x; 2.4190x over previous
import functools

import jax
import jax.numpy as jnp
from jax import lax
from jax.experimental import pallas as pl
from jax.experimental.pallas import tpu as pltpu
from jax.experimental.pallas import tpu_sc as plsc

D = 32
SEQ = 200
VOCAB = 1000000
BLK = 128
NBUF_A = 4
NBUF_B = 8
_INFO = plsc.get_sparse_core_info()
NW = _INFO.num_cores * _INFO.num_subcores
BPW = (VOCAB // BLK) // NW
MAIN = NW * BPW * BLK
TAIL = VOCAB - MAIN

_MESH = plsc.VectorSubcoreMesh(core_axis_name="c", subcore_axis_name="s")


def _wid():
  return lax.axis_index("s") * _INFO.num_cores + lax.axis_index("c")


def _t16(rows, iota, perms):
  cur = list(rows)
  for k in (8, 4, 2, 1):
    p_l, p_r = perms[k]
    mask = (iota & k) == 0
    nxt = [None] * 16
    for i in range(16):
      if i & k:
        continue
      j = i + k
      a, b = cur[i], cur[j]
      nxt[i] = jnp.where(mask, a, jnp.take(b, p_r))
      nxt[j] = jnp.where(mask, jnp.take(a, p_l), b)
    cur = nxt
  return cur


def _mk_perms(iota):
  return {k: ((iota + k) % 16, (iota - k) % 16) for k in (8, 4, 2, 1)}


@functools.partial(
    pl.kernel,
    out_type=jax.ShapeDtypeStruct((VOCAB * D,), jnp.float32),
    mesh=_MESH,
    compiler_params=pltpu.CompilerParams(use_tc_tiling_on_sc=True),
    scratch_types=[
        pltpu.VMEM((NBUF_A, D, BLK), jnp.float32),
        pltpu.VMEM((NBUF_A * D * BLK,), jnp.float32),
        pltpu.VMEM((TAIL, D), jnp.float32),
        pltpu.SemaphoreType.DMA((NBUF_A,)),
        pltpu.SemaphoreType.DMA((NBUF_A,)),
    ],
)
def _transpose_table(tab_t_hbm, tail_hbm, flat_hbm, in_v, out_v, tail_v,
                     gsem, wsem):
  wid = _wid()
  blk0 = wid * BPW
  iota = lax.iota(jnp.int32, 16)
  perms = _mk_perms(iota)

  def in_refs(i, buf):
    v0 = (blk0 + i) * BLK
    return tab_t_hbm.at[:, pl.ds(v0, BLK)], in_v.at[buf]

  def out_refs(i, buf):
    v0 = (blk0 + i) * BLK
    return out_v.at[pl.ds(buf * (D * BLK), D * BLK)], flat_hbm.at[
        pl.ds(v0 * D, D * BLK)]

  for b in range(NBUF_A):
    src, dst = in_refs(b, b)
    pltpu.async_copy(src, dst, gsem.at[b])

  @pl.loop(0, BPW)
  def body(i):
    buf = lax.rem(i, NBUF_A)
    src, dst = in_refs(i, buf)
    pltpu.make_async_copy(src, dst, gsem.at[buf]).wait()

    base = buf * (D * BLK)
    for cg in range(BLK // 16):
      for dg in range(D // 16):
        rows = [in_v[buf, dg * 16 + d, pl.ds(cg * 16, 16)] for d in range(16)]
        cols = _t16(rows, iota, perms)
        for c in range(16):
          out_v[pl.ds(base + (cg * 16 + c) * D + dg * 16, 16)] = cols[c]

    @pl.when(i + NBUF_A < BPW)
    def _():
      src2, dst2 = in_refs(i + NBUF_A, buf)
      pltpu.async_copy(src2, dst2, gsem.at[buf])

    @pl.when(i >= NBUF_A)
    def _():
      s3, d3 = out_refs(i - NBUF_A, buf)
      pltpu.make_async_copy(s3, d3, wsem.at[buf]).wait()

    s4, d4 = out_refs(i, buf)
    pltpu.async_copy(s4, d4, wsem.at[buf])

  for b in range(NBUF_A):
    i = BPW - NBUF_A + b
    s5, d5 = out_refs(i, lax.rem(jnp.int32(i), NBUF_A))
    pltpu.make_async_copy(s5, d5, wsem.at[lax.rem(jnp.int32(i), NBUF_A)]).wait()

  @pl.when(wid == NW - 1)
  def _():
    pltpu.sync_copy(tail_hbm, tail_v)
    nch = TAIL // 64

    @pl.loop(0, nch)
    def tail_body(ch):
      for t in range(64):
        r = ch * 64 + t
        out_v[pl.ds(t * D, 16)] = tail_v[r, pl.ds(0, 16)]
        out_v[pl.ds(t * D + 16, 16)] = tail_v[r, pl.ds(16, 16)]
      pltpu.sync_copy(out_v.at[pl.ds(0, 64 * D)],
                      flat_hbm.at[pl.ds(MAIN * D + ch * (64 * D), 64 * D)])


def _make_gather(batch: int):
  b_per_w = batch // NW

  @functools.partial(
      pl.kernel,
      out_type=jax.ShapeDtypeStruct((SEQ, D, batch), jnp.float32),
      mesh=_MESH,
      compiler_params=pltpu.CompilerParams(use_tc_tiling_on_sc=False),
      scratch_types=[
          pltpu.VMEM((SEQ, b_per_w), jnp.int32),
          pltpu.VMEM((NBUF_B, b_per_w, D), jnp.float32),
          pltpu.VMEM((NBUF_B, D, b_per_w), jnp.float32),
          pltpu.SemaphoreType.DMA((NBUF_B,)),
          pltpu.SemaphoreType.DMA((NBUF_B,)),
      ],
  )
  def gather_kernel(tab_hbm, tok_t_hbm, out_hbm, idx_v, rows_v, tmat_v,
                    gsem, wsem):
    wid = _wid()
    b0 = wid * b_per_w
    iota = lax.iota(jnp.int32, 16)
    perms = _mk_perms(iota)

    pltpu.sync_copy(tok_t_hbm.at[:, pl.ds(b0, b_per_w)], idx_v)

    def gather_refs(s, buf):
      return tab_hbm.at[idx_v.at[s]], rows_v.at[buf]

    def write_refs(s, buf):
      return tmat_v.at[buf], out_hbm.at[s, :, pl.ds(b0, b_per_w)]

    for b in range(NBUF_B):
      src, dst = gather_refs(b, b)
      pltpu.async_copy(src, dst, gsem.at[b])

    @pl.loop(0, SEQ)
    def body(s):
      buf = lax.rem(s, NBUF_B)
      src, dst = gather_refs(s, buf)
      pltpu.make_async_copy(src, dst, gsem.at[buf]).wait()

      for tg in range(b_per_w // 16):
        for dg in range(D // 16):
          rows = [rows_v[buf, tg * 16 + t, pl.ds(dg * 16, 16)]
                  for t in range(16)]
          cols = _t16(rows, iota, perms)
          for d in range(16):
            tmat_v[buf, dg * 16 + d, pl.ds(tg * 16, 16)] = cols[d]

      @pl.when(s + NBUF_B < SEQ)
      def _():
        src2, dst2 = gather_refs(s + NBUF_B, buf)
        pltpu.async_copy(src2, dst2, gsem.at[buf])

      @pl.when(s >= NBUF_B)
      def _():
        s3, d3 = write_refs(s - NBUF_B, buf)
        pltpu.make_async_copy(s3, d3, wsem.at[buf]).wait()

      s4, d4 = write_refs(s, buf)
      pltpu.async_copy(s4, d4, wsem.at[buf])

    for b in range(NBUF_B):
      s = SEQ - NBUF_B + b
      s5, d5 = write_refs(s, b)
      pltpu.make_async_copy(s5, d5, wsem.at[b]).wait()

  return gather_kernel


def kernel(news_tokens, embedding_table):
  batch, seq = news_tokens.shape
  tok_t = news_tokens.astype(jnp.int32).T
  tab_t = embedding_table.T
  tail = embedding_table[MAIN:]
  flat = _transpose_table(tab_t, tail)
  tab_lin = flat.reshape(VOCAB, D)
  out_t = _make_gather(batch)(tab_lin, tok_t)
  return jnp.transpose(out_t, (2, 0, 1))


# speedup vs baseline: 3.2123x; 1.3280x over previous
import functools

import jax
import jax.numpy as jnp
from jax import lax
from jax.experimental import pallas as pl
from jax.experimental.pallas import tpu as pltpu
from jax.experimental.pallas import tpu_sc as plsc

D = 32
SEQ = 200
VOCAB = 1000000
BLK = 128
NBUF_A = 4
NBUF_B = 8
_INFO = plsc.get_sparse_core_info()
NW = _INFO.num_cores * _INFO.num_subcores
BPW = (VOCAB // BLK) // NW
MAIN = NW * BPW * BLK
TAIL = VOCAB - MAIN

_MESH = plsc.VectorSubcoreMesh(core_axis_name="c", subcore_axis_name="s")


def _wid():
  return lax.axis_index("s") * _INFO.num_cores + lax.axis_index("c")


def _t16(rows, iota, perms):
  cur = list(rows)
  for k in (8, 4, 2, 1):
    p_l, p_r = perms[k]
    mask = (iota & k) == 0
    nxt = [None] * 16
    for i in range(16):
      if i & k:
        continue
      j = i + k
      a, b = cur[i], cur[j]
      nxt[i] = jnp.where(mask, a, jnp.take(b, p_r))
      nxt[j] = jnp.where(mask, jnp.take(a, p_l), b)
    cur = nxt
  return cur


def _mk_perms(iota):
  return {k: ((iota + k) % 16, (iota - k) % 16) for k in (8, 4, 2, 1)}


@functools.partial(
    pl.kernel,
    out_type=jax.ShapeDtypeStruct((VOCAB * D,), jnp.float32),
    mesh=_MESH,
    compiler_params=pltpu.CompilerParams(use_tc_tiling_on_sc=True),
    scratch_types=[
        pltpu.VMEM((NBUF_A, D, BLK), jnp.float32),
        pltpu.VMEM((NBUF_A * D * BLK,), jnp.float32),
        pltpu.VMEM((TAIL, D), jnp.float32),
        pltpu.SemaphoreType.DMA((NBUF_A,)),
        pltpu.SemaphoreType.DMA((NBUF_A,)),
    ],
)
def _transpose_table(tab_t_hbm, tail_hbm, flat_hbm, in_v, out_v, tail_v,
                     gsem, wsem):
  wid = _wid()
  blk0 = wid * BPW
  iota = lax.iota(jnp.int32, 16)
  perms = _mk_perms(iota)

  def in_refs(i, buf):
    v0 = (blk0 + i) * BLK
    return tab_t_hbm.at[:, pl.ds(v0, BLK)], in_v.at[buf]

  def out_refs(i, buf):
    v0 = (blk0 + i) * BLK
    return out_v.at[pl.ds(buf * (D * BLK), D * BLK)], flat_hbm.at[
        pl.ds(v0 * D, D * BLK)]

  for b in range(NBUF_A):
    src, dst = in_refs(b, b)
    pltpu.async_copy(src, dst, gsem.at[b])

  @pl.loop(0, BPW)
  def body(i):
    buf = lax.rem(i, NBUF_A)
    src, dst = in_refs(i, buf)
    pltpu.make_async_copy(src, dst, gsem.at[buf]).wait()

    base = buf * (D * BLK)
    for cg in range(BLK // 16):
      for dg in range(D // 16):
        rows = [in_v[buf, dg * 16 + d, pl.ds(cg * 16, 16)] for d in range(16)]
        cols = _t16(rows, iota, perms)
        for c in range(16):
          out_v[pl.ds(base + (cg * 16 + c) * D + dg * 16, 16)] = cols[c]

    @pl.when(i + NBUF_A < BPW)
    def _():
      src2, dst2 = in_refs(i + NBUF_A, buf)
      pltpu.async_copy(src2, dst2, gsem.at[buf])

    @pl.when(i >= NBUF_A)
    def _():
      s3, d3 = out_refs(i - NBUF_A, buf)
      pltpu.make_async_copy(s3, d3, wsem.at[buf]).wait()

    s4, d4 = out_refs(i, buf)
    pltpu.async_copy(s4, d4, wsem.at[buf])

  for b in range(NBUF_A):
    i = BPW - NBUF_A + b
    s5, d5 = out_refs(i, lax.rem(jnp.int32(i), NBUF_A))
    pltpu.make_async_copy(s5, d5, wsem.at[lax.rem(jnp.int32(i), NBUF_A)]).wait()

  @pl.when(wid == NW - 1)
  def _():
    pltpu.sync_copy(tail_hbm, tail_v)
    nch = TAIL // 64

    @pl.loop(0, nch)
    def tail_body(ch):
      for t in range(64):
        r = ch * 64 + t
        out_v[pl.ds(t * D, 16)] = tail_v[r, pl.ds(0, 16)]
        out_v[pl.ds(t * D + 16, 16)] = tail_v[r, pl.ds(16, 16)]
      pltpu.sync_copy(out_v.at[pl.ds(0, 64 * D)],
                      flat_hbm.at[pl.ds(MAIN * D + ch * (64 * D), 64 * D)])


def _make_gather(batch: int):
  b_per_w = batch // NW

  @functools.partial(
      pl.kernel,
      out_type=jax.ShapeDtypeStruct((SEQ, D // 8, batch // 128, 8, 128),
                                    jnp.float32),
      mesh=_MESH,
      compiler_params=pltpu.CompilerParams(use_tc_tiling_on_sc=False),
      scratch_types=[
          pltpu.VMEM((SEQ, b_per_w), jnp.int32),
          pltpu.VMEM((NBUF_B, b_per_w, D), jnp.float32),
          pltpu.VMEM((NBUF_B, D // 8, 8, b_per_w), jnp.float32),
          pltpu.SemaphoreType.DMA((NBUF_B,)),
          pltpu.SemaphoreType.DMA((NBUF_B,)),
      ],
  )
  def gather_kernel(tab_hbm, tok_t_hbm, out_hbm, idx_v, rows_v, tmat_v,
                    gsem, wsem):
    wid = _wid()
    b0 = wid * b_per_w
    iota = lax.iota(jnp.int32, 16)
    perms = _mk_perms(iota)

    pltpu.sync_copy(tok_t_hbm.at[:, pl.ds(b0, b_per_w)], idx_v)

    def gather_refs(s, buf):
      return tab_hbm.at[idx_v.at[s]], rows_v.at[buf]

    def write_refs(s, buf):
      return tmat_v.at[buf], out_hbm.at[s, :, wid]

    for b in range(NBUF_B):
      src, dst = gather_refs(b, b)
      pltpu.async_copy(src, dst, gsem.at[b])

    @pl.loop(0, SEQ)
    def body(s):
      buf = lax.rem(s, NBUF_B)
      src, dst = gather_refs(s, buf)
      pltpu.make_async_copy(src, dst, gsem.at[buf]).wait()

      for tg in range(b_per_w // 16):
        for dg in range(D // 16):
          rows = [rows_v[buf, tg * 16 + t, pl.ds(dg * 16, 16)]
                  for t in range(16)]
          cols = _t16(rows, iota, perms)
          for d in range(16):
            dd = dg * 16 + d
            tmat_v[buf, dd // 8, dd % 8, pl.ds(tg * 16, 16)] = cols[d]

      @pl.when(s + NBUF_B < SEQ)
      def _():
        src2, dst2 = gather_refs(s + NBUF_B, buf)
        pltpu.async_copy(src2, dst2, gsem.at[buf])

      @pl.when(s >= NBUF_B)
      def _():
        s3, d3 = write_refs(s - NBUF_B, buf)
        pltpu.make_async_copy(s3, d3, wsem.at[buf]).wait()

      s4, d4 = write_refs(s, buf)
      pltpu.async_copy(s4, d4, wsem.at[buf])

    for b in range(NBUF_B):
      s = SEQ - NBUF_B + b
      s5, d5 = write_refs(s, b)
      pltpu.make_async_copy(s5, d5, wsem.at[b]).wait()

  return gather_kernel


def kernel(news_tokens, embedding_table):
  batch, seq = news_tokens.shape
  tok_t = news_tokens.astype(jnp.int32).T
  tab_t = embedding_table.T
  tail = embedding_table[MAIN:]
  flat = _transpose_table(tab_t, tail)
  tab_lin = flat.reshape(VOCAB, D)
  out5 = _make_gather(batch)(tab_lin, tok_t)
  return jnp.transpose(out5, (2, 4, 0, 1, 3)).reshape(batch, seq, D)
